# Initial kernel scaffold; baseline (speedup 1.0000x reference)
import functools

import jax
import jax.numpy as jnp
import numpy as np
from jax import lax
from jax.experimental import pallas as pl
from jax.experimental.pallas import tpu as pltpu
from jax.experimental.pallas import tpu_sc as plsc

N = 10000
E = 320000
D = 128
DE = 16
H = 128
AUGW = 32
NC = 2
NS = 16
NPAD = 10240
RPT = NPAD // NS
EPT = E // (NC * NS)
CH = 128
NFULL = EPT // CH
TAIL = EPT - NFULL * CH
RB = 1024
GRID = (N + RB - 1) // RB


def _step(w):
    ncols = w.shape[1]
    nrm = jnp.sqrt(jnp.sum(w * w, axis=1, keepdims=True))
    nrm = np.float32(1e-4) + nrm * np.float32(1.0 / np.sqrt(ncols))
    return w / (nrm * np.float32(np.sqrt(ncols)))


def _prep_body(w0, wl1, wr1, wl2, wr2, wf,
               w0e, a1, b1, r1, a2, b2, r2, wfe):
    c_cat = np.sqrt((H + DE) / 0.5)
    ca = np.float32(c_cat / np.sqrt(H) * 0.5 / np.sqrt(2.0))
    cb = np.float32(c_cat / np.sqrt(DE) * 0.5 / np.sqrt(2.0))
    cr = np.float32(1.0 / np.sqrt(2.0))
    w0e[...] = _step(w0[...])
    wle = _step(_step(wl1[...]))
    a1[...] = ca * wle[:, :H]
    b1[...] = cb * wle[:, H:]
    r1[...] = cr * _step(_step(wr1[...]))
    wle2 = _step(_step(wl2[...]))
    a2[...] = ca * wle2[:, :H]
    b2[...] = cb * wle2[:, H:]
    r2[...] = cr * _step(_step(wr2[...]))
    wfe[...] = _step(wf[...])


def _prep_weights(W0, Wl1, Wr1, Wl2, Wr2, Wf):
    f32 = jnp.float32
    return pl.pallas_call(
        _prep_body,
        out_shape=(
            jax.ShapeDtypeStruct((H, D), f32),
            jax.ShapeDtypeStruct((H, H), f32),
            jax.ShapeDtypeStruct((H, DE), f32),
            jax.ShapeDtypeStruct((H, H), f32),
            jax.ShapeDtypeStruct((H, H), f32),
            jax.ShapeDtypeStruct((H, DE), f32),
            jax.ShapeDtypeStruct((H, H), f32),
            jax.ShapeDtypeStruct((H, H), f32),
        ),
    )(W0, Wl1, Wr1, Wl2, Wr2, Wf)


def _dot_t(x, w):
    return lax.dot_general(x, w, (((1,), (1,)), ((), ())),
                           preferred_element_type=jnp.float32)


def _mm_body(x_ref, w_ref, o_ref):
    o_ref[...] = _dot_t(x_ref[...], w_ref[...])


def _matmul(x, w):
    return pl.pallas_call(
        _mm_body,
        grid=(GRID,),
        in_specs=[pl.BlockSpec((RB, x.shape[1]), lambda i: (i, 0)),
                  pl.BlockSpec(w.shape, lambda i: (0, 0))],
        out_specs=pl.BlockSpec((RB, w.shape[0]), lambda i: (i, 0)),
        out_shape=jax.ShapeDtypeStruct((N, w.shape[0]), jnp.float32),
    )(x, w)


def _epi_body(sxa, sxb, ea, eb, h_ref, a_ref, b_ref, r_ref, o_ref):
    aug = ea[...] + eb[...]
    cnt = jnp.maximum(aug[:, DE:DE + 1], 1.0)
    mx = (sxa[...] + sxb[...]) / cnt
    me = aug[:, :DE] / cnt
    o = _dot_t(mx, a_ref[...]) + _dot_t(me, b_ref[...]) + _dot_t(h_ref[...], r_ref[...])
    nrm = jnp.maximum(jnp.sqrt(jnp.sum(o * o, axis=1, keepdims=True)), 1e-12)
    o_ref[...] = o / nrm


def _epilogue(sxa, sxb, ea, eb, h, a, b, r):
    return pl.pallas_call(
        _epi_body,
        grid=(GRID,),
        in_specs=[pl.BlockSpec((RB, H), lambda i: (i, 0)),
                  pl.BlockSpec((RB, H), lambda i: (i, 0)),
                  pl.BlockSpec((RB, AUGW), lambda i: (i, 0)),
                  pl.BlockSpec((RB, AUGW), lambda i: (i, 0)),
                  pl.BlockSpec((RB, H), lambda i: (i, 0)),
                  pl.BlockSpec((H, H), lambda i: (0, 0)),
                  pl.BlockSpec((H, DE), lambda i: (0, 0)),
                  pl.BlockSpec((H, H), lambda i: (0, 0))],
        out_specs=pl.BlockSpec((RB, H), lambda i: (i, 0)),
        out_shape=jax.ShapeDtypeStruct((N, H), jnp.float32),
    )(sxa, sxb, ea, eb, h, a, b, r)


_sc_mesh = plsc.VectorSubcoreMesh(core_axis_name="c", subcore_axis_name="s")


def _zero_rows(rows, width):
    def body(r, _):
        for j in range(width // 16):
            rows[r, pl.ds(j * 16, 16)] = jnp.zeros((16,), jnp.float32)
        return 0
    lax.fori_loop(0, CH, body, 0)


@functools.partial(
    pl.kernel,
    mesh=_sc_mesh,
    out_type=jax.ShapeDtypeStruct((NC, NPAD, H), jnp.float32),
    scratch_types=[
        pltpu.VMEM((CH,), jnp.int32),
        pltpu.VMEM((CH,), jnp.int32),
        pltpu.VMEM((TAIL,), jnp.int32),
        pltpu.VMEM((TAIL,), jnp.int32),
        pltpu.VMEM((CH, H), jnp.float32),
        pltpu.VMEM_SHARED((NPAD, H), jnp.float32),
        pltpu.SemaphoreType.DMA,
    ],
)
def _sc_segsum(h_hbm, src_hbm, dst_hbm, out_hbm,
               sidx, didx, sidx_t, didx_t, rows, acc, sem):
    c = lax.axis_index("c")
    s = lax.axis_index("s")
    _zero_rows(rows, H)
    zbase = s * RPT
    for k in range(RPT // CH):
        pltpu.sync_copy(rows, acc.at[pl.ds(zbase + k * CH, CH)])
    plsc.subcore_barrier()
    ebase = c * (E // NC) + s * EPT

    def chunk(j, _):
        base = ebase + j * CH
        pltpu.sync_copy(src_hbm.at[pl.ds(base, CH)], sidx)
        pltpu.sync_copy(dst_hbm.at[pl.ds(base, CH)], didx)
        pltpu.async_copy(h_hbm.at[sidx], rows, sem).wait()
        pltpu.sync_copy(rows, acc.at[didx], add=True)
        return 0

    lax.fori_loop(0, NFULL, chunk, 0)
    tbase = ebase + NFULL * CH
    pltpu.sync_copy(src_hbm.at[pl.ds(tbase, TAIL)], sidx_t)
    pltpu.sync_copy(dst_hbm.at[pl.ds(tbase, TAIL)], didx_t)
    pltpu.async_copy(h_hbm.at[sidx_t], rows.at[pl.ds(0, TAIL)], sem).wait()
    pltpu.sync_copy(rows.at[pl.ds(0, TAIL)], acc.at[didx_t], add=True)
    plsc.subcore_barrier()
    for k in range(RPT // CH):
        off = zbase + k * CH
        pltpu.sync_copy(acc.at[pl.ds(off, CH)], rows)
        pltpu.sync_copy(rows, out_hbm.at[c].at[pl.ds(off, CH)])


@functools.partial(
    pl.kernel,
    mesh=_sc_mesh,
    out_type=jax.ShapeDtypeStruct((NC, NPAD, AUGW), jnp.float32),
    scratch_types=[
        pltpu.VMEM((CH,), jnp.int32),
        pltpu.VMEM((TAIL,), jnp.int32),
        pltpu.VMEM((CH, AUGW), jnp.float32),
        pltpu.VMEM_SHARED((NPAD, AUGW), jnp.float32),
    ],
)
def _sc_edgeprep(ea_hbm, dst_hbm, out_hbm, didx, didx_t, rows, acc):
    c = lax.axis_index("c")
    s = lax.axis_index("s")
    _zero_rows(rows, AUGW)
    zbase = s * RPT
    for k in range(RPT // CH):
        pltpu.sync_copy(rows, acc.at[pl.ds(zbase + k * CH, CH)])
    plsc.subcore_barrier()
    ebase = c * (E // NC) + s * EPT

    def chunk(j, _):
        base = ebase + j * CH
        pltpu.sync_copy(ea_hbm.at[pl.ds(base, CH)], rows)
        pltpu.sync_copy(dst_hbm.at[pl.ds(base, CH)], didx)
        pltpu.sync_copy(rows, acc.at[didx], add=True)
        return 0

    lax.fori_loop(0, NFULL, chunk, 0)
    tbase = ebase + NFULL * CH
    pltpu.sync_copy(ea_hbm.at[pl.ds(tbase, TAIL)], rows.at[pl.ds(0, TAIL)])
    pltpu.sync_copy(dst_hbm.at[pl.ds(tbase, TAIL)], didx_t)
    pltpu.sync_copy(rows.at[pl.ds(0, TAIL)], acc.at[didx_t], add=True)
    plsc.subcore_barrier()
    for k in range(RPT // CH):
        off = zbase + k * CH
        pltpu.sync_copy(acc.at[pl.ds(off, CH)], rows)
        pltpu.sync_copy(rows, out_hbm.at[c].at[pl.ds(off, CH)])


def kernel(x, edge_index, edge_attr, W0, Wl1, Wr1, Wl2, Wr2, Wf):
    src = edge_index[0]
    dst = edge_index[1]
    ea_aug = jnp.concatenate(
        [edge_attr,
         jnp.ones((E, 1), jnp.float32),
         jnp.zeros((E, AUGW - DE - 1), jnp.float32)], axis=1)
    w0e, a1, b1, r1, a2, b2, r2, wfe = _prep_weights(W0, Wl1, Wr1, Wl2, Wr2, Wf)
    eparts = _sc_edgeprep(ea_aug, dst)
    h = _matmul(x, w0e)
    for (a, b, r) in ((a1, b1, r1), (a2, b2, r2)):
        sx = _sc_segsum(h, src, dst)
        h = _epilogue(sx[0], sx[1], eparts[0], eparts[1], h, a, b, r)
    return _matmul(h, wfe)


# speedup vs baseline: 5.4378x; 5.4378x over previous
import functools

import jax
import jax.numpy as jnp
import numpy as np
from jax import lax
from jax.experimental import pallas as pl
from jax.experimental.pallas import tpu as pltpu
from jax.experimental.pallas import tpu_sc as plsc

N = 10000
E = 320000
D = 128
DE = 16
H = 128
AUGW = 128
NC = 2
NS = 16
NPAD = 10240
RPT = NPAD // NS
EPT = E // (NC * NS)
CH = 128
NFULL = EPT // CH
TAIL = EPT - NFULL * CH
RB = 1024
GRID = (N + RB - 1) // RB


def _step(w):
    ncols = w.shape[1]
    nrm = jnp.sqrt(jnp.sum(w * w, axis=1, keepdims=True))
    nrm = np.float32(1e-4) + nrm * np.float32(1.0 / np.sqrt(ncols))
    return w / (nrm * np.float32(np.sqrt(ncols)))


def _prep_body(w0, wl1, wr1, wl2, wr2, wf,
               w0e, a1, b1, r1, a2, b2, r2, wfe):
    c_cat = np.sqrt((H + DE) / 0.5)
    ca = np.float32(c_cat / np.sqrt(H) * 0.5 / np.sqrt(2.0))
    cb = np.float32(c_cat / np.sqrt(DE) * 0.5 / np.sqrt(2.0))
    cr = np.float32(1.0 / np.sqrt(2.0))
    w0e[...] = _step(w0[...])
    wle = _step(_step(wl1[...]))
    a1[...] = ca * wle[:, :H]
    b1[...] = cb * wle[:, H:]
    r1[...] = cr * _step(_step(wr1[...]))
    wle2 = _step(_step(wl2[...]))
    a2[...] = ca * wle2[:, :H]
    b2[...] = cb * wle2[:, H:]
    r2[...] = cr * _step(_step(wr2[...]))
    wfe[...] = _step(wf[...])


def _prep_weights(W0, Wl1, Wr1, Wl2, Wr2, Wf):
    f32 = jnp.float32
    return pl.pallas_call(
        _prep_body,
        out_shape=(
            jax.ShapeDtypeStruct((H, D), f32),
            jax.ShapeDtypeStruct((H, H), f32),
            jax.ShapeDtypeStruct((H, DE), f32),
            jax.ShapeDtypeStruct((H, H), f32),
            jax.ShapeDtypeStruct((H, H), f32),
            jax.ShapeDtypeStruct((H, DE), f32),
            jax.ShapeDtypeStruct((H, H), f32),
            jax.ShapeDtypeStruct((H, H), f32),
        ),
    )(W0, Wl1, Wr1, Wl2, Wr2, Wf)


def _dot_t(x, w):
    return lax.dot_general(x, w, (((1,), (1,)), ((), ())),
                           preferred_element_type=jnp.float32)


def _mm_body(x_ref, w_ref, o_ref):
    o_ref[...] = _dot_t(x_ref[...], w_ref[...])


def _matmul(x, w):
    return pl.pallas_call(
        _mm_body,
        grid=(GRID,),
        in_specs=[pl.BlockSpec((RB, x.shape[1]), lambda i: (i, 0)),
                  pl.BlockSpec(w.shape, lambda i: (0, 0))],
        out_specs=pl.BlockSpec((RB, w.shape[0]), lambda i: (i, 0)),
        out_shape=jax.ShapeDtypeStruct((N, w.shape[0]), jnp.float32),
    )(x, w)


def _epi_body(sxa, sxb, ea, eb, h_ref, a_ref, b_ref, r_ref, o_ref):
    aug = ea[...] + eb[...]
    cnt = jnp.maximum(aug[:, DE:DE + 1], 1.0)
    mx = (sxa[...] + sxb[...]) / cnt
    me = aug[:, :DE] / cnt
    o = _dot_t(mx, a_ref[...]) + _dot_t(me, b_ref[...]) + _dot_t(h_ref[...], r_ref[...])
    nrm = jnp.maximum(jnp.sqrt(jnp.sum(o * o, axis=1, keepdims=True)), 1e-12)
    o_ref[...] = o / nrm


def _epilogue(sxa, sxb, ea, eb, h, a, b, r):
    return pl.pallas_call(
        _epi_body,
        grid=(GRID,),
        in_specs=[pl.BlockSpec((RB, H), lambda i: (i, 0)),
                  pl.BlockSpec((RB, H), lambda i: (i, 0)),
                  pl.BlockSpec((RB, AUGW), lambda i: (i, 0)),
                  pl.BlockSpec((RB, AUGW), lambda i: (i, 0)),
                  pl.BlockSpec((RB, H), lambda i: (i, 0)),
                  pl.BlockSpec((H, H), lambda i: (0, 0)),
                  pl.BlockSpec((H, DE), lambda i: (0, 0)),
                  pl.BlockSpec((H, H), lambda i: (0, 0))],
        out_specs=pl.BlockSpec((RB, H), lambda i: (i, 0)),
        out_shape=jax.ShapeDtypeStruct((N, H), jnp.float32),
    )(sxa, sxb, ea, eb, h, a, b, r)


_sc_mesh = plsc.VectorSubcoreMesh(core_axis_name="c", subcore_axis_name="s")


def _zero_rows(rows, width):
    def body(r, _):
        for j in range(width // 16):
            rows[r, pl.ds(j * 16, 16)] = jnp.zeros((16,), jnp.float32)
        return 0
    lax.fori_loop(0, CH, body, 0)


@functools.partial(
    pl.kernel,
    mesh=_sc_mesh,
    out_type=jax.ShapeDtypeStruct((NC, NPAD, H), jnp.float32),
    scratch_types=[
        pltpu.VMEM((CH,), jnp.int32),
        pltpu.VMEM((CH,), jnp.int32),
        pltpu.VMEM((TAIL,), jnp.int32),
        pltpu.VMEM((TAIL,), jnp.int32),
        pltpu.VMEM((CH, H), jnp.float32),
        pltpu.VMEM_SHARED((NPAD, H), jnp.float32),
        pltpu.SemaphoreType.DMA,
    ],
)
def _sc_segsum(h_hbm, src_hbm, dst_hbm, out_hbm,
               sidx, didx, sidx_t, didx_t, rows, acc, sem):
    c = lax.axis_index("c")
    s = lax.axis_index("s")
    _zero_rows(rows, H)
    zbase = s * RPT
    for k in range(RPT // CH):
        pltpu.sync_copy(rows, acc.at[pl.ds(zbase + k * CH, CH)])
    plsc.subcore_barrier()
    ebase = c * (E // NC) + s * EPT

    def chunk(j, _):
        base = ebase + j * CH
        pltpu.sync_copy(src_hbm.at[pl.ds(base, CH)], sidx)
        pltpu.sync_copy(dst_hbm.at[pl.ds(base, CH)], didx)
        pltpu.async_copy(h_hbm.at[sidx], rows, sem).wait()
        pltpu.sync_copy(rows, acc.at[didx], add=True)
        return 0

    lax.fori_loop(0, NFULL, chunk, 0)
    tbase = ebase + NFULL * CH
    pltpu.sync_copy(src_hbm.at[pl.ds(tbase, TAIL)], sidx_t)
    pltpu.sync_copy(dst_hbm.at[pl.ds(tbase, TAIL)], didx_t)
    pltpu.async_copy(h_hbm.at[sidx_t], rows.at[pl.ds(0, TAIL)], sem).wait()
    pltpu.sync_copy(rows.at[pl.ds(0, TAIL)], acc.at[didx_t], add=True)
    plsc.subcore_barrier()
    for k in range(RPT // CH):
        off = zbase + k * CH
        pltpu.sync_copy(acc.at[pl.ds(off, CH)], rows)
        pltpu.sync_copy(rows, out_hbm.at[c].at[pl.ds(off, CH)])


@functools.partial(
    pl.kernel,
    mesh=_sc_mesh,
    out_type=jax.ShapeDtypeStruct((NC, NPAD, AUGW), jnp.float32),
    scratch_types=[
        pltpu.VMEM((CH,), jnp.int32),
        pltpu.VMEM((TAIL,), jnp.int32),
        pltpu.VMEM((CH, AUGW), jnp.float32),
        pltpu.VMEM_SHARED((NPAD, AUGW), jnp.float32),
    ],
)
def _sc_edgeprep(ea_hbm, dst_hbm, out_hbm, didx, didx_t, rows, acc):
    c = lax.axis_index("c")
    s = lax.axis_index("s")
    _zero_rows(rows, AUGW)
    zbase = s * RPT
    for k in range(RPT // CH):
        pltpu.sync_copy(rows, acc.at[pl.ds(zbase + k * CH, CH)])
    plsc.subcore_barrier()
    ebase = c * (E // NC) + s * EPT

    def chunk(j, _):
        base = ebase + j * CH
        pltpu.sync_copy(ea_hbm.at[pl.ds(base, CH)], rows)
        pltpu.sync_copy(dst_hbm.at[pl.ds(base, CH)], didx)
        pltpu.sync_copy(rows, acc.at[didx], add=True)
        return 0

    lax.fori_loop(0, NFULL, chunk, 0)
    tbase = ebase + NFULL * CH
    pltpu.sync_copy(ea_hbm.at[pl.ds(tbase, TAIL)], rows.at[pl.ds(0, TAIL)])
    pltpu.sync_copy(dst_hbm.at[pl.ds(tbase, TAIL)], didx_t)
    pltpu.sync_copy(rows.at[pl.ds(0, TAIL)], acc.at[didx_t], add=True)
    plsc.subcore_barrier()
    for k in range(RPT // CH):
        off = zbase + k * CH
        pltpu.sync_copy(acc.at[pl.ds(off, CH)], rows)
        pltpu.sync_copy(rows, out_hbm.at[c].at[pl.ds(off, CH)])


def kernel(x, edge_index, edge_attr, W0, Wl1, Wr1, Wl2, Wr2, Wf):
    src = edge_index[0]
    dst = edge_index[1]
    ea_aug = jnp.concatenate(
        [edge_attr,
         jnp.ones((E, 1), jnp.float32),
         jnp.zeros((E, AUGW - DE - 1), jnp.float32)], axis=1)
    w0e, a1, b1, r1, a2, b2, r2, wfe = _prep_weights(W0, Wl1, Wr1, Wl2, Wr2, Wf)
    eparts = _sc_edgeprep(ea_aug, dst)
    h = _matmul(x, w0e)
    for (a, b, r) in ((a1, b1, r1), (a2, b2, r2)):
        sx = _sc_segsum(h, src, dst)
        h = _epilogue(sx[0], sx[1], eparts[0], eparts[1], h, a, b, r)
    return _matmul(h, wfe)
